# Initial kernel scaffold; baseline (speedup 1.0000x reference)
import jax
import jax.numpy as jnp
from jax.experimental import pallas as pl


def kernel(input_ids, e0, e1, mask_token_embed):
    raise NotImplementedError("write your pallas kernel here")


# speedup vs baseline: 1.4993x; 1.4993x over previous
import functools

import jax
import jax.numpy as jnp
from jax import lax
from jax.experimental import pallas as pl
from jax.experimental.pallas import tpu as pltpu
from jax.experimental.pallas import tpu_sc as plsc

FVS = 512
MASK_ID = FVS * FVS
D = 1024
L = 16


def _sc_lookup(ids, t0, t1, n_workers, chunk):
    n = ids.shape[0]
    per_w = n // n_workers
    n_chunks = per_w // chunk
    mesh = plsc.VectorSubcoreMesh(
        core_axis_name="c", subcore_axis_name="s", num_cores=2, num_subcores=16)

    @functools.partial(
        pl.kernel,
        out_type=jax.ShapeDtypeStruct((n, D), jnp.float32),
        mesh=mesh,
        scratch_types=[
            pltpu.VMEM((per_w,), jnp.int32),
            pltpu.VMEM((chunk,), jnp.int32),
            pltpu.VMEM((chunk,), jnp.int32),
            pltpu.VMEM((chunk, D), jnp.float32),
            pltpu.VMEM((chunk, D), jnp.float32),
            pltpu.VMEM((chunk, D), jnp.float32),
            pltpu.SemaphoreType.DMA,
            pltpu.SemaphoreType.DMA,
        ],
    )
    def k(ids_hbm, t0_hbm, t1_hbm, out_hbm,
          ids_v, idx0_v, idx1_v, rows0_v, rows1_v, out_v, sem0, sem1):
        n_cores = lax.axis_size("c")
        wid = lax.axis_index("s") * n_cores + lax.axis_index("c")
        base = wid * per_w
        pltpu.sync_copy(ids_hbm.at[pl.ds(base, per_w)], ids_v)

        def chunk_body(c, carry):
            tok0 = c * chunk
            for j in range(chunk // L):
                v = ids_v[pl.ds(tok0 + j * L, L)]
                m = v == MASK_ID
                idx0_v[pl.ds(j * L, L)] = jnp.where(m, FVS, v & (FVS - 1))
                idx1_v[pl.ds(j * L, L)] = jnp.where(m, FVS, (v >> 9) & (FVS - 1))
            cp0 = pltpu.async_copy(t0_hbm.at[idx0_v], rows0_v, sem0)
            cp1 = pltpu.async_copy(t1_hbm.at[idx1_v], rows1_v, sem1)
            cp0.wait()
            cp1.wait()

            def add_body(t, tc):
                for kk in range(D // L):
                    s = pl.ds(kk * L, L)
                    out_v[t, s] = rows0_v[t, s] + rows1_v[t, s]
                return tc
            lax.fori_loop(0, chunk, add_body, 0, unroll=False)
            pltpu.sync_copy(out_v, out_hbm.at[pl.ds(base + tok0, chunk)])
            return carry

        lax.fori_loop(0, n_chunks, chunk_body, 0, unroll=False)

    return k(ids, t0, t1)


def kernel(input_ids, e0, e1, mask_token_embed):
    orig_shape = input_ids.shape
    ids = input_ids.reshape(-1).astype(jnp.int32)
    t0 = jnp.concatenate([e0, mask_token_embed], axis=0)
    t1 = jnp.concatenate([e1, jnp.zeros_like(mask_token_embed)], axis=0)
    out = _sc_lookup(ids, t0, t1, n_workers=32, chunk=32)
    return out.reshape(orig_shape + (D,))


# speedup vs baseline: 2.4824x; 1.6557x over previous
import functools

import jax
import jax.numpy as jnp
from jax import lax
from jax.experimental import pallas as pl
from jax.experimental.pallas import tpu as pltpu
from jax.experimental.pallas import tpu_sc as plsc

FVS = 512
MASK_ID = FVS * FVS
D = 1024
L = 16
NBUF = 2


def _sc_lookup(ids, t0, t1, n_workers, chunk):
    n = ids.shape[0]
    per_w = n // n_workers
    n_chunks = per_w // chunk
    n_pairs = n_chunks // NBUF
    mesh = plsc.VectorSubcoreMesh(
        core_axis_name="c", subcore_axis_name="s", num_cores=2, num_subcores=16)

    @functools.partial(
        pl.kernel,
        out_type=jax.ShapeDtypeStruct((n, D), jnp.float32),
        mesh=mesh,
        scratch_types=[
            pltpu.VMEM((per_w,), jnp.int32),
            [pltpu.VMEM((chunk,), jnp.int32) for _ in range(NBUF)],
            [pltpu.VMEM((chunk,), jnp.int32) for _ in range(NBUF)],
            [pltpu.VMEM((chunk, D), jnp.float32) for _ in range(NBUF)],
            [pltpu.VMEM((chunk, D), jnp.float32) for _ in range(NBUF)],
            [pltpu.VMEM((chunk, D), jnp.float32) for _ in range(NBUF)],
            [pltpu.SemaphoreType.DMA for _ in range(NBUF)],
            [pltpu.SemaphoreType.DMA for _ in range(NBUF)],
        ],
    )
    def k(ids_hbm, t0_hbm, t1_hbm, out_hbm,
          ids_v, idx0, idx1, r0, r1, outb, gsem, ssem):
        n_cores = lax.axis_size("c")
        wid = lax.axis_index("s") * n_cores + lax.axis_index("c")
        base = wid * per_w
        pltpu.sync_copy(ids_hbm.at[pl.ds(base, per_w)], ids_v)

        def compute_idx(c, b):
            tok0 = c * chunk
            for j in range(chunk // L):
                v = ids_v[pl.ds(tok0 + j * L, L)]
                m = v == MASK_ID
                idx0[b][pl.ds(j * L, L)] = jnp.where(m, FVS, v & (FVS - 1))
                idx1[b][pl.ds(j * L, L)] = jnp.where(m, FVS, (v >> 9) & (FVS - 1))

        def start_gather(b):
            pltpu.async_copy(t0_hbm.at[idx0[b]], r0[b], gsem[b])
            pltpu.async_copy(t1_hbm.at[idx1[b]], r1[b], gsem[b])

        def wait_gather(b):
            pltpu.make_async_copy(t0_hbm.at[idx0[b]], r0[b], gsem[b]).wait()
            pltpu.make_async_copy(t1_hbm.at[idx1[b]], r1[b], gsem[b]).wait()

        def add(b):
            def body(t, carry):
                for kk in range(D // L):
                    s = pl.ds(kk * L, L)
                    outb[b][t, s] = r0[b][t, s] + r1[b][t, s]
                return carry
            lax.fori_loop(0, chunk, body, 0, unroll=False)

        def start_scatter(c, b):
            pltpu.async_copy(
                outb[b], out_hbm.at[pl.ds(base + c * chunk, chunk)], ssem[b])

        def wait_scatter(b):
            pltpu.make_async_copy(
                outb[b], out_hbm.at[pl.ds(base, chunk)], ssem[b]).wait()

        for b in range(NBUF):
            compute_idx(b, b)
            start_gather(b)

        def pair_body(i, carry):
            for b in range(NBUF):
                c = NBUF * i + b
                wait_gather(b)

                @pl.when(i >= 1)
                def _():
                    wait_scatter(b)

                add(b)
                start_scatter(c, b)

                @pl.when(c + NBUF < n_chunks)
                def _():
                    compute_idx(c + NBUF, b)
                    start_gather(b)
            return carry

        lax.fori_loop(0, n_pairs, pair_body, 0, unroll=False)
        for b in range(NBUF):
            wait_scatter(b)

    return k(ids, t0, t1)


def kernel(input_ids, e0, e1, mask_token_embed):
    orig_shape = input_ids.shape
    ids = input_ids.reshape(-1).astype(jnp.int32)
    t0 = jnp.concatenate([e0, mask_token_embed], axis=0)
    t1 = jnp.concatenate([e1, jnp.zeros_like(mask_token_embed)], axis=0)
    out = _sc_lookup(ids, t0, t1, n_workers=32, chunk=16)
    return out.reshape(orig_shape + (D,))
